# baseline (speedup 1.0000x reference)
import jax
import jax.numpy as jnp
from jax.experimental import pallas as pl
from jax.experimental.pallas import tpu as pltpu
from jax.experimental.pallas import tpu_sc as plsc

_B = 16384
_D = 16
_R = 5
_SLICE = _B // 32


def _sc_body(zut_hbm, zit_hbm, pr_hbm, prm_hbm, puit_hbm, xui_hbm,
             zu_v, zi_v, pr_v, prm_v, po_v, xo_v, sem):
    c = jax.lax.axis_index("c")
    s = jax.lax.axis_index("s")
    b0 = (c * 16 + s) * _SLICE
    pltpu.async_copy(zut_hbm.at[:, pl.ds(b0, _SLICE)], zu_v, sem).wait()
    pltpu.async_copy(zit_hbm.at[:, pl.ds(b0, _SLICE)], zi_v, sem).wait()
    pltpu.async_copy(pr_hbm, pr_v, sem).wait()
    pltpu.async_copy(prm_hbm, prm_v, sem).wait()

    prow = [pr_v[pl.ds(i * _D, _D)] for i in range(2 * _D)]
    pv = prm_v[...]

    @pl.loop(0, _SLICE // 16)
    def _chunk(ci):
        base = ci * 16
        zs = [zu_v[j, pl.ds(base, 16)] for j in range(_D)]
        t0 = jnp.zeros((16,), jnp.float32)
        t1 = jnp.zeros((16,), jnp.float32)
        for k in range(_D):
            zik = zi_v[k, pl.ds(base, 16)]
            u0 = zs[0] * prow[0][k]
            u1 = zs[0] * prow[_D][k]
            for j in range(1, _D):
                u0 = u0 + zs[j] * prow[j][k]
                u1 = u1 + zs[j] * prow[_D + j][k]
            t0 = t0 + u0 * zik
            t1 = t1 + u1 * zik
        ps = [pv[r] * t0 + pv[_R + r] * t1 for r in range(_R)]
        m = ps[0]
        for r in range(1, _R):
            m = jnp.maximum(m, ps[r])
        es = [jnp.exp(p - m) for p in ps]
        den = es[0]
        num = es[0] * pv[2 * _R]
        for r in range(1, _R):
            den = den + es[r]
            num = num + es[r] * pv[2 * _R + r]
        x = num / den
        for r in range(_R):
            po_v[r, pl.ds(base, 16)] = ps[r]
        xo_v[pl.ds(base, 16)] = x

    pltpu.async_copy(po_v, puit_hbm.at[:, pl.ds(b0, _SLICE)], sem).wait()
    pltpu.async_copy(xo_v, xui_hbm.at[pl.ds(b0, _SLICE)], sem).wait()


def kernel(zu, zi, P, A, relations):
    b, d = zu.shape
    r = relations.shape[0]
    zut = zu.T
    zit = zi.T
    pr = P.reshape(2 * d * d)
    at = jnp.transpose(A, (1, 2, 0)).reshape(2 * r)
    prm = jnp.concatenate(
        [at, relations, jnp.zeros((1,), jnp.float32)])
    mesh = plsc.VectorSubcoreMesh(core_axis_name="c", subcore_axis_name="s")
    sck = pl.kernel(
        _sc_body,
        out_type=[
            jax.ShapeDtypeStruct((r, b), jnp.float32),
            jax.ShapeDtypeStruct((b,), jnp.float32),
        ],
        mesh=mesh,
        scratch_types=[
            pltpu.VMEM((d, _SLICE), jnp.float32),
            pltpu.VMEM((d, _SLICE), jnp.float32),
            pltpu.VMEM((2 * d * d,), jnp.float32),
            pltpu.VMEM((d,), jnp.float32),
            pltpu.VMEM((r, _SLICE), jnp.float32),
            pltpu.VMEM((_SLICE,), jnp.float32),
            pltpu.SemaphoreType.DMA,
        ],
    )
    puit, xui = sck(zut, zit, pr, prm)
    return (xui, puit.T)
